# Initial kernel scaffold; baseline (speedup 1.0000x reference)
import jax
import jax.numpy as jnp
from jax.experimental import pallas as pl


def kernel(x, senders, receivers, edge_feature, emb, Wu, bu, Wv, bv):
    raise NotImplementedError("write your pallas kernel here")


# speedup vs baseline: 1.6282x; 1.6282x over previous
import functools

import jax
import jax.numpy as jnp
from jax import lax
from jax.experimental import pallas as pl
from jax.experimental.pallas import tpu as pltpu
from jax.experimental.pallas import tpu_sc as plsc

N_NODES = 10000
N_EDGES = 320000
D = 128
NUM_TYPES = 4

NC = 2
NS = 16
NW = NC * NS
LANES = 16

CHUNK = 256
N_CHUNKS = N_EDGES // CHUNK


def _project_body(x_ref, wu_ref, bu_ref, wv_ref, bv_ref, emb_ref,
                  xu_ref, xvt_ref):
    xb = x_ref[...]
    du = jnp.dot(xb, wu_ref[...], preferred_element_type=jnp.float32) + bu_ref[...]
    dv = jnp.dot(xb, wv_ref[...], preferred_element_type=jnp.float32) + bv_ref[...]
    xu_ref[...] = du
    emb = emb_ref[...]
    for t in range(NUM_TYPES):
        xvt_ref[t] = dv * emb[t][None, :]


def _project(x, wu, bu2, wv, bv2, emb):
    R = 2000
    grid = (N_NODES // R,)
    return pl.pallas_call(
        _project_body,
        grid=grid,
        in_specs=[
            pl.BlockSpec((R, D), lambda i: (i, 0)),
            pl.BlockSpec((D, D), lambda i: (0, 0)),
            pl.BlockSpec((1, D), lambda i: (0, 0)),
            pl.BlockSpec((D, D), lambda i: (0, 0)),
            pl.BlockSpec((1, D), lambda i: (0, 0)),
            pl.BlockSpec((NUM_TYPES, D), lambda i: (0, 0)),
        ],
        out_specs=[
            pl.BlockSpec((R, D), lambda i: (i, 0)),
            pl.BlockSpec((NUM_TYPES, R, D), lambda i: (0, i, 0)),
        ],
        out_shape=[
            jax.ShapeDtypeStruct((N_NODES, D), jnp.float32),
            jax.ShapeDtypeStruct((NUM_TYPES, N_NODES, D), jnp.float32),
        ],
    )(x, wu, bu2, wv, bv2, emb)


def _edge_body(xu_hbm, xvt_hbm, s_hbm, r_hbm, e_hbm, out_hbm,
               sidx0, sidx1, vidx0, vidx1, rbuf, ebuf, u_buf, v_buf,
               out_buf, sem):
    cid = lax.axis_index("c")
    sid = lax.axis_index("s")
    wid = sid * NC + cid
    n_mine = (N_CHUNKS - wid + NW - 1) // NW
    lanes = lax.iota(jnp.int32, LANES)

    def chunk_body(k, carry):
        c = wid + k * NW
        base = c * CHUNK
        pltpu.sync_copy(s_hbm.at[pl.ds(base, 128)], sidx0)
        pltpu.sync_copy(s_hbm.at[pl.ds(base + 128, 128)], sidx1)
        pltpu.sync_copy(r_hbm.at[pl.ds(base, CHUNK)], rbuf)
        pltpu.sync_copy(e_hbm.at[pl.ds(base, CHUNK)], ebuf)
        for i in range(8):
            sl = pl.ds(i * LANES, LANES)
            vidx0[sl] = ebuf[sl] * N_NODES + rbuf[sl]
        for i in range(8):
            sl = pl.ds(i * LANES, LANES)
            sl_hi = pl.ds(128 + i * LANES, LANES)
            vidx1[sl] = ebuf[sl_hi] * N_NODES + rbuf[sl_hi]
        cps = [
            pltpu.async_copy(xu_hbm.at[sidx0], u_buf.at[pl.ds(0, 128)], sem),
            pltpu.async_copy(xu_hbm.at[sidx1], u_buf.at[pl.ds(128, 128)], sem),
            pltpu.async_copy(xvt_hbm.at[vidx0], v_buf.at[pl.ds(0, 128)], sem),
            pltpu.async_copy(xvt_hbm.at[vidx1], v_buf.at[pl.ds(128, 128)], sem),
        ]
        for cp in cps:
            cp.wait()

        def group_body(g, carry2):
            rows = g * LANES + lanes

            def d_body(db, acc):
                for t in range(LANES):
                    col = jnp.full((LANES,), db * LANES + t, jnp.int32)
                    u = plsc.load_gather(u_buf, [rows, col])
                    v = plsc.load_gather(v_buf, [rows, col])
                    acc = acc + u * v
                return acc

            acc = lax.fori_loop(0, D // LANES, d_body,
                                jnp.zeros((LANES,), jnp.float32))
            out_buf[pl.ds(g * LANES, LANES)] = acc
            return carry2

        lax.fori_loop(0, CHUNK // LANES, group_body, 0)
        pltpu.sync_copy(out_buf, out_hbm.at[pl.ds(base, CHUNK)])
        return carry

    lax.fori_loop(0, n_mine, chunk_body, 0)


@functools.cache
def _edge_kernel():
    return pl.kernel(
        _edge_body,
        out_type=jax.ShapeDtypeStruct((N_EDGES,), jnp.float32),
        mesh=plsc.VectorSubcoreMesh(core_axis_name="c", subcore_axis_name="s"),
        scratch_types=[
            pltpu.VMEM((128,), jnp.int32),
            pltpu.VMEM((128,), jnp.int32),
            pltpu.VMEM((128,), jnp.int32),
            pltpu.VMEM((128,), jnp.int32),
            pltpu.VMEM((CHUNK,), jnp.int32),
            pltpu.VMEM((CHUNK,), jnp.int32),
            pltpu.VMEM((CHUNK, D), jnp.float32),
            pltpu.VMEM((CHUNK, D), jnp.float32),
            pltpu.VMEM((CHUNK,), jnp.float32),
            pltpu.SemaphoreType.DMA,
        ],
        compiler_params=pltpu.CompilerParams(needs_layout_passes=False),
    )


def kernel(x, senders, receivers, edge_feature, emb, Wu, bu, Wv, bv):
    xu, xvt = _project(x, Wu, bu.reshape(1, D), Wv, bv.reshape(1, D), emb)
    xvt_flat = xvt.reshape(NUM_TYPES * N_NODES, D)
    return _edge_kernel()(xu, xvt_flat, senders, receivers, edge_feature)


# speedup vs baseline: 1.9193x; 1.1788x over previous
import functools

import jax
import jax.numpy as jnp
from jax import lax
from jax.experimental import pallas as pl
from jax.experimental.pallas import tpu as pltpu
from jax.experimental.pallas import tpu_sc as plsc

N_NODES = 10000
N_EDGES = 320000
D = 128
NUM_TYPES = 4

NC = 2
NS = 16
NW = NC * NS
LANES = 16

E_PER_W = N_EDGES // NW
CHUNK = 80
N_CHUNKS = E_PER_W // CHUNK
GROUPS = CHUNK // LANES


def _project_body(x_ref, wu_ref, bu_ref, wv_ref, bv_ref, emb_ref,
                  xu_ref, xvt_ref):
    xb = x_ref[...]
    du = jnp.dot(xb, wu_ref[...], preferred_element_type=jnp.float32) + bu_ref[...]
    dv = jnp.dot(xb, wv_ref[...], preferred_element_type=jnp.float32) + bv_ref[...]
    xu_ref[...] = du
    emb = emb_ref[...]
    for t in range(NUM_TYPES):
        xvt_ref[t] = dv * emb[t][None, :]


def _project(x, wu, bu2, wv, bv2, emb):
    R = 2000
    grid = (N_NODES // R,)
    return pl.pallas_call(
        _project_body,
        grid=grid,
        in_specs=[
            pl.BlockSpec((R, D), lambda i: (i, 0)),
            pl.BlockSpec((D, D), lambda i: (0, 0)),
            pl.BlockSpec((1, D), lambda i: (0, 0)),
            pl.BlockSpec((D, D), lambda i: (0, 0)),
            pl.BlockSpec((1, D), lambda i: (0, 0)),
            pl.BlockSpec((NUM_TYPES, D), lambda i: (0, 0)),
        ],
        out_specs=[
            pl.BlockSpec((R, D), lambda i: (i, 0)),
            pl.BlockSpec((NUM_TYPES, R, D), lambda i: (0, i, 0)),
        ],
        out_shape=[
            jax.ShapeDtypeStruct((N_NODES, D), jnp.float32),
            jax.ShapeDtypeStruct((NUM_TYPES, N_NODES, D), jnp.float32),
        ],
    )(x, wu, bu2, wv, bv2, emb)


def _edge_body(xu_hbm, xvt_hbm, s_hbm, r_hbm, e_hbm, out_hbm,
               sbuf, vbuf, ebuf, out_all,
               u0, v0, u1, v1, sem0, sem1):
    cid = lax.axis_index("c")
    sid = lax.axis_index("s")
    wid = sid * NC + cid
    ebase = pl.multiple_of(wid * E_PER_W, E_PER_W)
    lanes = lax.iota(jnp.int32, LANES)

    pltpu.sync_copy(s_hbm.at[pl.ds(ebase, E_PER_W)], sbuf)
    pltpu.sync_copy(r_hbm.at[pl.ds(ebase, E_PER_W)], vbuf)
    pltpu.sync_copy(e_hbm.at[pl.ds(ebase, E_PER_W)], ebuf)

    def fuse_body(i, carry):
        sl = pl.ds(i * LANES, LANES)
        vbuf[sl] = ebuf[sl] * N_NODES + vbuf[sl]
        return carry

    lax.fori_loop(0, E_PER_W // LANES, fuse_body, 0)

    def gathers(k, ub, vb, sem):
        off = pl.multiple_of(k * CHUNK, CHUNK)
        pltpu.async_copy(xu_hbm.at[sbuf.at[pl.ds(off, CHUNK)]], ub, sem)
        pltpu.async_copy(xvt_hbm.at[vbuf.at[pl.ds(off, CHUNK)]], vb, sem)

    def drain(ub, vb, sem):
        pltpu.make_async_copy(xu_hbm.at[sbuf.at[pl.ds(0, CHUNK)]], ub, sem).wait()
        pltpu.make_async_copy(xvt_hbm.at[vbuf.at[pl.ds(0, CHUNK)]], vb, sem).wait()

    def compute(k, ub, vb):
        def group_body(g, carry2):
            rows = g * LANES + lanes

            def d_body(db, acc):
                for t in range(LANES):
                    col = jnp.full((LANES,), db * LANES + t, jnp.int32)
                    u = plsc.load_gather(ub, [rows, col])
                    v = plsc.load_gather(vb, [rows, col])
                    acc = acc + u * v
                return acc

            acc = lax.fori_loop(0, D // LANES, d_body,
                                jnp.zeros((LANES,), jnp.float32))
            out_all[pl.ds(k * CHUNK + g * LANES, LANES)] = acc
            return carry2

        lax.fori_loop(0, GROUPS, group_body, 0)

    gathers(0, u0, v0, sem0)

    def chunk_pair(p, carry):
        k = p * 2
        gathers(k + 1, u1, v1, sem1)
        drain(u0, v0, sem0)
        compute(k, u0, v0)

        @pl.when(k + 2 < N_CHUNKS)
        def _():
            gathers(k + 2, u0, v0, sem0)

        drain(u1, v1, sem1)
        compute(k + 1, u1, v1)
        return carry

    lax.fori_loop(0, (N_CHUNKS - 1) // 2, chunk_pair, 0)
    drain(u0, v0, sem0)
    compute(N_CHUNKS - 1, u0, v0)

    pltpu.sync_copy(out_all, out_hbm.at[pl.ds(ebase, E_PER_W)])


@functools.cache
def _edge_kernel():
    return pl.kernel(
        _edge_body,
        out_type=jax.ShapeDtypeStruct((N_EDGES,), jnp.float32),
        mesh=plsc.VectorSubcoreMesh(core_axis_name="c", subcore_axis_name="s"),
        scratch_types=[
            pltpu.VMEM((E_PER_W,), jnp.int32),
            pltpu.VMEM((E_PER_W,), jnp.int32),
            pltpu.VMEM((E_PER_W,), jnp.int32),
            pltpu.VMEM((E_PER_W,), jnp.float32),
            pltpu.VMEM((CHUNK, D), jnp.float32),
            pltpu.VMEM((CHUNK, D), jnp.float32),
            pltpu.VMEM((CHUNK, D), jnp.float32),
            pltpu.VMEM((CHUNK, D), jnp.float32),
            pltpu.SemaphoreType.DMA,
            pltpu.SemaphoreType.DMA,
        ],
        compiler_params=pltpu.CompilerParams(needs_layout_passes=False),
    )


def kernel(x, senders, receivers, edge_feature, emb, Wu, bu, Wv, bv):
    xu, xvt = _project(x, Wu, bu.reshape(1, D), Wv, bv.reshape(1, D), emb)
    xvt_flat = xvt.reshape(NUM_TYPES * N_NODES, D)
    return _edge_kernel()(xu, xvt_flat, senders, receivers, edge_feature)


# speedup vs baseline: 2.6051x; 1.3573x over previous
import functools

import jax
import jax.numpy as jnp
from jax import lax
from jax.experimental import pallas as pl
from jax.experimental.pallas import tpu as pltpu
from jax.experimental.pallas import tpu_sc as plsc

N_NODES = 10000
N_EDGES = 320000
D = 128
NUM_TYPES = 4

NC = 2
NS = 16
NW = NC * NS
LANES = 16

D_PER_W = D // NW
CHUNK = 2000
N_CHUNKS = N_EDGES // CHUNK
GROUPS = CHUNK // LANES


def _project_body(x_ref, wu_ref, bu_ref, wv_ref, bv_ref,
                  xut_ref, xvt_ref):
    xb = x_ref[...]
    du = jnp.dot(xb, wu_ref[...], preferred_element_type=jnp.float32) + bu_ref[...]
    dv = jnp.dot(xb, wv_ref[...], preferred_element_type=jnp.float32) + bv_ref[...]
    xut_ref[...] = du.T
    xvt_ref[...] = dv.T


def _project(x, wu, bu2, wv, bv2):
    R = N_NODES
    grid = (N_NODES // R,)
    return pl.pallas_call(
        _project_body,
        grid=grid,
        in_specs=[
            pl.BlockSpec((R, D), lambda i: (i, 0)),
            pl.BlockSpec((D, D), lambda i: (0, 0)),
            pl.BlockSpec((1, D), lambda i: (0, 0)),
            pl.BlockSpec((D, D), lambda i: (0, 0)),
            pl.BlockSpec((1, D), lambda i: (0, 0)),
        ],
        out_specs=[
            pl.BlockSpec((D, R), lambda i: (0, i)),
            pl.BlockSpec((D, R), lambda i: (0, i)),
        ],
        out_shape=[
            jax.ShapeDtypeStruct((D, N_NODES), jnp.float32),
            jax.ShapeDtypeStruct((D, N_NODES), jnp.float32),
        ],
    )(x, wu, bu2, wv, bv2)


def _reduce_body(part_ref, out_ref):
    out_ref[...] = jnp.sum(part_ref[...], axis=0, keepdims=True)


def _reduce(partials):
    B = 12800
    grid = (N_EDGES // B,)
    return pl.pallas_call(
        _reduce_body,
        grid=grid,
        in_specs=[pl.BlockSpec((NW, B), lambda i: (0, i))],
        out_specs=pl.BlockSpec((1, B), lambda i: (0, i)),
        out_shape=jax.ShapeDtypeStruct((1, N_EDGES), jnp.float32),
    )(partials)


def _edge_body(xut_hbm, xvt_hbm, emb_hbm, s_hbm, r_hbm, e_hbm, part_hbm,
               utbl, vtbl, etbl,
               s0, r0, e0, p0, s1, r1, e1, p1, sem0, sem1, osem):
    cid = lax.axis_index("c")
    sid = lax.axis_index("s")
    wid = sid * NC + cid
    dbase = pl.multiple_of(wid * D_PER_W, D_PER_W)
    lanes = lax.iota(jnp.int32, LANES)

    tbase = pl.multiple_of(dbase * N_NODES, D_PER_W * N_NODES)
    pltpu.sync_copy(xut_hbm.at[pl.ds(tbase, D_PER_W * N_NODES)], utbl)
    pltpu.sync_copy(xvt_hbm.at[pl.ds(tbase, D_PER_W * N_NODES)], vtbl)
    pltpu.sync_copy(emb_hbm, etbl)

    def loads(k, sb, rb, eb, sem):
        off = pl.multiple_of(k * CHUNK, CHUNK)
        pltpu.async_copy(s_hbm.at[pl.ds(off, CHUNK)], sb, sem)
        pltpu.async_copy(r_hbm.at[pl.ds(off, CHUNK)], rb, sem)
        pltpu.async_copy(e_hbm.at[pl.ds(off, CHUNK)], eb, sem)

    def drain(sb, rb, eb, sem):
        pltpu.make_async_copy(s_hbm.at[pl.ds(0, CHUNK)], sb, sem).wait()
        pltpu.make_async_copy(r_hbm.at[pl.ds(0, CHUNK)], rb, sem).wait()
        pltpu.make_async_copy(e_hbm.at[pl.ds(0, CHUNK)], eb, sem).wait()

    def compute(sb, rb, eb, pb):
        def group_body(g, carry2):
            sl = pl.ds(g * LANES, LANES)
            sv = sb[sl]
            rv = rb[sl]
            ev = eb[sl] * D

            def one_d(dl, acc):
                u = plsc.load_gather(utbl, [sv + dl * N_NODES])
                v = plsc.load_gather(vtbl, [rv + dl * N_NODES])
                w = plsc.load_gather(etbl, [ev + (dbase + dl)])
                return acc + u * v * w

            acc = jnp.zeros((LANES,), jnp.float32)
            for dl in range(D_PER_W):
                acc = one_d(dl, acc)
            pb[sl] = acc
            return carry2

        lax.fori_loop(0, GROUPS, group_body, 0)

    def store(k, pb):
        off = pl.multiple_of(wid * N_EDGES + k * CHUNK, CHUNK)
        pltpu.async_copy(pb, part_hbm.at[pl.ds(off, CHUNK)], osem)

    def store_wait(pb):
        pltpu.make_async_copy(pb, part_hbm.at[pl.ds(0, CHUNK)], osem).wait()

    loads(0, s0, r0, e0, sem0)

    def chunk_pair(p, carry):
        k = p * 2
        loads(k + 1, s1, r1, e1, sem1)
        drain(s0, r0, e0, sem0)

        @pl.when(p > 0)
        def _():
            store_wait(p0)

        compute(s0, r0, e0, p0)
        store(k, p0)

        @pl.when(k + 2 < N_CHUNKS)
        def _():
            loads(k + 2, s0, r0, e0, sem0)

        drain(s1, r1, e1, sem1)

        @pl.when(p > 0)
        def _():
            store_wait(p1)

        compute(s1, r1, e1, p1)
        store(k + 1, p1)
        return carry

    lax.fori_loop(0, N_CHUNKS // 2, chunk_pair, 0)
    store_wait(p0)
    store_wait(p1)


@functools.cache
def _edge_kernel():
    return pl.kernel(
        _edge_body,
        out_type=jax.ShapeDtypeStruct((NW * N_EDGES,), jnp.float32),
        mesh=plsc.VectorSubcoreMesh(core_axis_name="c", subcore_axis_name="s"),
        scratch_types=[
            pltpu.VMEM((D_PER_W * N_NODES,), jnp.float32),
            pltpu.VMEM((D_PER_W * N_NODES,), jnp.float32),
            pltpu.VMEM((NUM_TYPES * D,), jnp.float32),
            pltpu.VMEM((CHUNK,), jnp.int32),
            pltpu.VMEM((CHUNK,), jnp.int32),
            pltpu.VMEM((CHUNK,), jnp.int32),
            pltpu.VMEM((CHUNK,), jnp.float32),
            pltpu.VMEM((CHUNK,), jnp.int32),
            pltpu.VMEM((CHUNK,), jnp.int32),
            pltpu.VMEM((CHUNK,), jnp.int32),
            pltpu.VMEM((CHUNK,), jnp.float32),
            pltpu.SemaphoreType.DMA,
            pltpu.SemaphoreType.DMA,
            pltpu.SemaphoreType.DMA,
        ],
        compiler_params=pltpu.CompilerParams(needs_layout_passes=False),
    )


def kernel(x, senders, receivers, edge_feature, emb, Wu, bu, Wv, bv):
    xut, xvt = _project(x, Wu, bu.reshape(1, D), Wv, bv.reshape(1, D))
    partials = _edge_kernel()(xut.reshape(D * N_NODES), xvt.reshape(D * N_NODES),
                              emb.reshape(NUM_TYPES * D),
                              senders, receivers, edge_feature)
    return _reduce(partials.reshape(NW, N_EDGES)).reshape(N_EDGES)


# speedup vs baseline: 2.9487x; 1.1319x over previous
import functools

import jax
import jax.numpy as jnp
from jax import lax
from jax.experimental import pallas as pl
from jax.experimental.pallas import tpu as pltpu
from jax.experimental.pallas import tpu_sc as plsc

N_NODES = 10000
N_EDGES = 320000
D = 128
HALF = D // 2
NUM_TYPES = 4

NC = 2
NS = 16
NW = NC * NS
LANES = 16

P_PER_S = HALF // NS
E_PER_C = N_EDGES // NC
CHUNK = 1600
N_CHUNKS = E_PER_C // CHUNK
GROUPS = CHUNK // LANES


def _pack_pairs(d_f32):
    lo = lax.bitcast_convert_type(d_f32[:, :HALF].astype(jnp.bfloat16),
                                  jnp.uint16).astype(jnp.uint32)
    hi = lax.bitcast_convert_type(d_f32[:, HALF:].astype(jnp.bfloat16),
                                  jnp.uint16).astype(jnp.uint32)
    return lax.bitcast_convert_type(lo | (hi << 16), jnp.int32).T


def _project_body(x_ref, wu_ref, bu_ref, wv_ref, bv_ref, s_ref, r_ref, e_ref,
                  upk_ref, vpk_ref, pidx_ref):
    xb = x_ref[...]
    du = jnp.dot(xb, wu_ref[...], preferred_element_type=jnp.float32) + bu_ref[...]
    dv = jnp.dot(xb, wv_ref[...], preferred_element_type=jnp.float32) + bv_ref[...]
    upk_ref[...] = _pack_pairs(du)
    vpk_ref[...] = _pack_pairs(dv)
    pidx_ref[...] = s_ref[...] | (r_ref[...] << 14) | (e_ref[...] << 28)


def _project(x, wu, bu2, wv, bv2, s2d, r2d, e2d):
    whole = lambda shape: pl.BlockSpec(shape, lambda: tuple(0 for _ in shape))
    return pl.pallas_call(
        _project_body,
        grid=(),
        in_specs=[
            whole((N_NODES, D)),
            whole((D, D)),
            whole((1, D)),
            whole((D, D)),
            whole((1, D)),
            whole((N_EDGES // D, D)),
            whole((N_EDGES // D, D)),
            whole((N_EDGES // D, D)),
        ],
        out_specs=[
            whole((HALF, N_NODES)),
            whole((HALF, N_NODES)),
            whole((N_EDGES // D, D)),
        ],
        out_shape=[
            jax.ShapeDtypeStruct((HALF, N_NODES), jnp.int32),
            jax.ShapeDtypeStruct((HALF, N_NODES), jnp.int32),
            jax.ShapeDtypeStruct((N_EDGES // D, D), jnp.int32),
        ],
    )(x, wu, bu2, wv, bv2, s2d, r2d, e2d)


def _reduce_body(part_ref, out_ref):
    out_ref[...] = jnp.sum(part_ref[...], axis=0, keepdims=True)


def _reduce(partials):
    B = 12800
    grid = (N_EDGES // B,)
    return pl.pallas_call(
        _reduce_body,
        grid=grid,
        in_specs=[pl.BlockSpec((NS, B), lambda i: (0, i))],
        out_specs=pl.BlockSpec((1, B), lambda i: (0, i)),
        out_shape=jax.ShapeDtypeStruct((1, N_EDGES), jnp.float32),
    )(partials)


def _edge_body(upk_hbm, vpk_hbm, emb_hbm, pidx_hbm, part_hbm,
               utbl, vtbl, etbl, sh_idx,
               i0, p0, i1, p1, sem0, sem1, osem):
    cid = lax.axis_index("c")
    sid = lax.axis_index("s")
    lanes = lax.iota(jnp.int32, LANES)

    tbase = pl.multiple_of(sid * (P_PER_S * N_NODES), P_PER_S * N_NODES)
    pltpu.sync_copy(upk_hbm.at[pl.ds(tbase, P_PER_S * N_NODES)], utbl)
    pltpu.sync_copy(vpk_hbm.at[pl.ds(tbase, P_PER_S * N_NODES)], vtbl)
    pltpu.sync_copy(emb_hbm, etbl)

    @pl.when(sid == 0)
    def _():
        hbase = pl.multiple_of(cid * E_PER_C, E_PER_C)
        pltpu.sync_copy(pidx_hbm.at[pl.ds(hbase, E_PER_C)], sh_idx)

    plsc.subcore_barrier()

    def loads(k, ib, sem):
        off = pl.multiple_of(k * CHUNK, CHUNK)
        pltpu.async_copy(sh_idx.at[pl.ds(off, CHUNK)], ib, sem)

    def drain(ib, sem):
        pltpu.make_async_copy(sh_idx.at[pl.ds(0, CHUNK)], ib, sem).wait()

    def compute(ib, pb):
        def group_body(g, carry2):
            sl = pl.ds(g * LANES, LANES)
            pk = ib[sl]
            sv = lax.bitwise_and(pk, 0x3FFF)
            rv = lax.bitwise_and(lax.shift_right_logical(pk, 14), 0x3FFF)
            ev = lax.shift_left(lax.shift_right_logical(pk, 28), 7)

            acc = jnp.zeros((LANES,), jnp.float32)
            for dl in range(P_PER_S):
                u = plsc.load_gather(utbl, [sv + dl * N_NODES])
                v = plsc.load_gather(vtbl, [rv + dl * N_NODES])
                prod = plsc.bitcast(u, jnp.bfloat16) * plsc.bitcast(v, jnp.bfloat16)
                plo, phi = plsc.unpack(prod, format=plsc.PackFormat.INTERLEAVED)
                drow = sid * P_PER_S + dl
                elo = plsc.load_gather(etbl, [ev + drow])
                ehi = plsc.load_gather(etbl, [ev + (drow + HALF)])
                acc = acc + plo * elo + phi * ehi
            pb[sl] = acc
            return carry2

        lax.fori_loop(0, GROUPS, group_body, 0)

    def store(k, pb):
        off = pl.multiple_of(sid * N_EDGES + cid * E_PER_C + k * CHUNK, CHUNK)
        pltpu.async_copy(pb, part_hbm.at[pl.ds(off, CHUNK)], osem)

    def store_wait(pb):
        pltpu.make_async_copy(pb, part_hbm.at[pl.ds(0, CHUNK)], osem).wait()

    loads(0, i0, sem0)

    def chunk_pair(p, carry):
        k = p * 2
        loads(k + 1, i1, sem1)
        drain(i0, sem0)

        @pl.when(p > 0)
        def _():
            store_wait(p0)

        compute(i0, p0)
        store(k, p0)

        @pl.when(k + 2 < N_CHUNKS)
        def _():
            loads(k + 2, i0, sem0)

        drain(i1, sem1)

        @pl.when(p > 0)
        def _():
            store_wait(p1)

        compute(i1, p1)
        store(k + 1, p1)
        return carry

    lax.fori_loop(0, N_CHUNKS // 2, chunk_pair, 0)
    store_wait(p0)
    store_wait(p1)


@functools.cache
def _edge_kernel():
    return pl.kernel(
        _edge_body,
        out_type=jax.ShapeDtypeStruct((NS * N_EDGES,), jnp.float32),
        mesh=plsc.VectorSubcoreMesh(core_axis_name="c", subcore_axis_name="s"),
        scratch_types=[
            pltpu.VMEM((P_PER_S * N_NODES,), jnp.int32),
            pltpu.VMEM((P_PER_S * N_NODES,), jnp.int32),
            pltpu.VMEM((NUM_TYPES * D,), jnp.float32),
            pltpu.VMEM_SHARED((E_PER_C,), jnp.int32),
            pltpu.VMEM((CHUNK,), jnp.int32),
            pltpu.VMEM((CHUNK,), jnp.float32),
            pltpu.VMEM((CHUNK,), jnp.int32),
            pltpu.VMEM((CHUNK,), jnp.float32),
            pltpu.SemaphoreType.DMA,
            pltpu.SemaphoreType.DMA,
            pltpu.SemaphoreType.DMA,
        ],
        compiler_params=pltpu.CompilerParams(needs_layout_passes=False),
    )


def kernel(x, senders, receivers, edge_feature, emb, Wu, bu, Wv, bv):
    upk, vpk, pidx = _project(
        x, Wu, bu.reshape(1, D), Wv, bv.reshape(1, D),
        senders.reshape(N_EDGES // D, D),
        receivers.reshape(N_EDGES // D, D),
        edge_feature.reshape(N_EDGES // D, D),
    )
    partials = _edge_kernel()(
        upk.reshape(HALF * N_NODES), vpk.reshape(HALF * N_NODES),
        emb.reshape(NUM_TYPES * D), pidx.reshape(N_EDGES))
    return _reduce(partials.reshape(NS, N_EDGES)).reshape(N_EDGES)
